# baseline (speedup 1.0000x reference)
import functools

import jax
import jax.numpy as jnp
from jax import lax
from jax.experimental import pallas as pl
from jax.experimental.pallas import tpu as pltpu
from jax.experimental.pallas import tpu_sc as plsc

_H = 224
_W = 224
_NH = 14
_S = _NH * _NH
_C = 96
_N = _H * _W
_BAND = 16 * _W
_NEG = -1e30
_TN = (((0,), (0,)), ((), ()))
_HI = lax.Precision.HIGHEST

_NWORK = 32
_BPW = _N // _NWORK
_CHUNK = _BPW // 2
_TD = 128


def _tc_body(x_ref, lab_ref, means_ref, centT, cnorm, acc, cnt):
    t = pl.program_id(0)
    ph = t // _NH
    c = t % _NH

    ri = lax.broadcasted_iota(jnp.int32, (_BAND, _S), 0)
    ji = lax.broadcasted_iota(jnp.int32, (_BAND, _S), 1)
    rf = ri.astype(jnp.float32)
    jf = ji.astype(jnp.float32)
    rb = jnp.floor(rf * (1.0 / 16.0))
    bx = rb - 14.0 * jnp.floor((rb + 0.5) * (1.0 / 14.0))
    sy = jnp.floor((jf + 0.5) * (1.0 / 14.0))
    sx = jf - 14.0 * sy
    cf = c.astype(jnp.float32)

    @pl.when((c == 0) & (ph == 1))
    def _():
        cn = acc[...]
        centT[...] = cn
        cnorm[...] = jnp.sum(cn * cn, axis=0, keepdims=True)

    @pl.when((c == 0) & ((ph == 2) | (ph == 3)))
    def _():
        cn = acc[...] / (cnt[...] + 1e-16)
        centT[...] = cn
        cnorm[...] = jnp.sum(cn * cn, axis=0, keepdims=True)

    @pl.when((c == 0) & (ph == 4))
    def _():
        means_ref[...] = acc[...] / jnp.maximum(cnt[...], 1.0)

    @pl.when((c == 0) & (ph < 4))
    def _():
        acc[...] = jnp.zeros_like(acc)
        cnt[...] = jnp.zeros_like(cnt)

    X = x_ref[pl.ds(c * _BAND, _BAND), :]

    @pl.when(ph == 0)
    def _():
        w = jnp.where((sy == cf) & (sx == bx), 1.0 / 256.0, 0.0)
        acc[...] += lax.dot_general(X, w, _TN,
                                    preferred_element_type=jnp.float32,
                                    precision=_HI)

    valid = (jnp.abs(sy - cf) <= 1.0) & (jnp.abs(sx - bx) <= 1.0)

    @pl.when((ph == 1) | (ph == 2))
    def _():
        lm = 2.0 * jnp.dot(X, centT[...],
                           preferred_element_type=jnp.float32,
                           precision=_HI) - cnorm[...]
        lm = jnp.where(valid, lm, _NEG)
        m = jnp.max(lm, axis=1, keepdims=True)
        e = jnp.exp(lm - m)
        aff = e / jnp.sum(e, axis=1, keepdims=True)
        acc[...] += lax.dot_general(X, aff, _TN,
                                    preferred_element_type=jnp.float32)
        cnt[...] += jnp.sum(aff, axis=0, keepdims=True)

    @pl.when(ph == 3)
    def _():
        lm = 2.0 * jnp.dot(X, centT[...],
                           preferred_element_type=jnp.float32,
                           precision=_HI) - cnorm[...]
        lm = jnp.where(valid, lm, _NEG)
        m = jnp.max(lm, axis=1, keepdims=True)
        candj = jnp.where(lm >= m, ji, _S)
        labj = jnp.min(candj, axis=1, keepdims=True)
        lab_ref[...] = labj
        onehot = (ji == labj).astype(jnp.float32)
        acc[...] += lax.dot_general(X, onehot, _TN,
                                    preferred_element_type=jnp.float32,
                                    precision=_HI)
        cnt[...] += jnp.sum(onehot, axis=0, keepdims=True)


def _tc_ssn(x_pix):
    return pl.pallas_call(
        _tc_body,
        grid=(4 * _NH + 1,),
        in_specs=[pl.BlockSpec((_N, _C), lambda t: (0, 0))],
        out_specs=[
            pl.BlockSpec((_BAND, 1),
                         lambda t: (jnp.where(t // _NH == 3, t % _NH, _NH), 0)),
            pl.BlockSpec((_C, _S), lambda t: (0, 0)),
        ],
        out_shape=[
            jax.ShapeDtypeStruct(((_NH + 1) * _BAND, 1), jnp.int32),
            jax.ShapeDtypeStruct((_C, _S), jnp.float32),
        ],
        scratch_shapes=[
            pltpu.VMEM((_C, _S), jnp.float32),
            pltpu.VMEM((1, _S), jnp.float32),
            pltpu.VMEM((_C, _S), jnp.float32),
            pltpu.VMEM((1, _S), jnp.float32),
        ],
        compiler_params=pltpu.CompilerParams(
            dimension_semantics=("arbitrary",),
            vmem_limit_bytes=60 * 1024 * 1024,
        ),
    )(x_pix)


def _sc_recolor(table, labels):
    mesh = plsc.VectorSubcoreMesh(core_axis_name="c", subcore_axis_name="s")

    @functools.partial(
        pl.kernel, mesh=mesh,
        out_type=jax.ShapeDtypeStruct((_N, _TD), jnp.float32),
        scratch_types=[
            pltpu.VMEM((_CHUNK,), jnp.int32),
            pltpu.VMEM((_CHUNK, _TD), jnp.float32),
            pltpu.SemaphoreType.DMA,
        ],
    )
    def k(table_hbm, lab_hbm, out_hbm, idx_v, rows_v, sem):
        wid = lax.axis_index("s") * 2 + lax.axis_index("c")
        base = wid * _BPW
        for j in range(_BPW // _CHUNK):
            off = base + j * _CHUNK
            pltpu.sync_copy(lab_hbm.at[pl.ds(off, _CHUNK)], idx_v)
            pltpu.async_copy(table_hbm.at[idx_v], rows_v, sem).wait()
            pltpu.sync_copy(rows_v, out_hbm.at[pl.ds(off, _CHUNK)])

    return k(table, labels)


def kernel(x, stoken_size):
    del stoken_size
    x_pix = x[0].reshape(_C, _N).T
    lab2, meansT = _tc_ssn(x_pix)
    labels = lab2[:_N, 0]
    table = jnp.pad(meansT.T, ((0, 0), (0, _TD - _C)))
    out = _sc_recolor(table, labels)
    return out[:, :_C].T.reshape(1, _C, _H, _W)
